# baseline (speedup 1.0000x reference)
import dataclasses
import functools

import jax
import jax.numpy as jnp
from jax import lax
from jax.experimental import pallas as pl
from jax.experimental.pallas import tpu as pltpu
from jax.experimental.pallas import tpu_sc as plsc

N = 50000
C = 128
KV = 27
P = 8192
TOT = KV * P
NC = 2
NS = 16
NW = NC * NS

G_ROWS = TOT // NW
G_CH = 128
G_NCH = G_ROWS // G_CH

TILE = 8352
TPC = 3
S_ROWS = TOT // NS
SEG = 1728
SEG_G = SEG // 16
CCH = 128
CAP = 5248
CAP_CH = CAP // CCH
DUMP = TILE
ACC_ROWS = 8448
WB_CH = 96
WB_N = TILE // WB_CH


def _gather_sc(feats, in_idx):
    mesh = plsc.VectorSubcoreMesh(core_axis_name="c", subcore_axis_name="s")

    @functools.partial(
        pl.kernel,
        out_type=jax.ShapeDtypeStruct((TOT, C), jnp.float32),
        mesh=mesh,
        scratch_types=[
            pltpu.VMEM((G_ROWS,), jnp.int32),
            pltpu.VMEM((G_CH, C), jnp.float32),
            pltpu.VMEM((G_CH, C), jnp.float32),
            pltpu.SemaphoreType.DMA,
            pltpu.SemaphoreType.DMA,
            pltpu.SemaphoreType.DMA,
            pltpu.SemaphoreType.DMA,
        ],
    )
    def k(feats_hbm, idx_hbm, out_hbm, idx_v, rows_a, rows_b,
          gsem_a, gsem_b, wsem_a, wsem_b):
        wid = lax.axis_index("s") * NC + lax.axis_index("c")
        base = wid * G_ROWS
        pltpu.sync_copy(idx_hbm.at[pl.ds(base, G_ROWS)], idx_v)

        def gather_desc(j, buf, sem):
            return pltpu.make_async_copy(
                feats_hbm.at[idx_v.at[pl.ds(j * G_CH, G_CH)]], buf, sem)

        def write_desc(j, buf, sem):
            return pltpu.make_async_copy(
                buf, out_hbm.at[pl.ds(base + j * G_CH, G_CH)], sem)

        gather_desc(0, rows_a, gsem_a).start()
        gather_desc(1, rows_b, gsem_b).start()

        def body(p, carry):
            j0 = 2 * p
            j1 = j0 + 1
            gather_desc(j0, rows_a, gsem_a).wait()
            write_desc(j0, rows_a, wsem_a).start()
            gather_desc(j1, rows_b, gsem_b).wait()
            write_desc(j1, rows_b, wsem_b).start()

            @pl.when(p + 1 < G_NCH // 2)
            def _():
                write_desc(j0, rows_a, wsem_a).wait()
                gather_desc(j0 + 2, rows_a, gsem_a).start()
                write_desc(j1, rows_b, wsem_b).wait()
                gather_desc(j1 + 2, rows_b, gsem_b).start()

            return carry

        lax.fori_loop(0, G_NCH // 2, body, jnp.int32(0))
        write_desc(G_NCH - 2, rows_a, wsem_a).wait()
        write_desc(G_NCH - 1, rows_b, wsem_b).wait()

    return k(feats, in_idx)


def _gemm_tc(gathered, w):
    BP = 4096

    def body(x_ref, w_ref, o_ref):
        x = x_ref[0].astype(jnp.bfloat16)
        wb = w_ref[0].astype(jnp.bfloat16)
        o_ref[...] = jnp.dot(x, wb, preferred_element_type=jnp.float32)[None]

    return pl.pallas_call(
        body,
        grid=(KV, P // BP),
        in_specs=[
            pl.BlockSpec((1, BP, C), lambda k, p: (k, p, 0)),
            pl.BlockSpec((1, C, C), lambda k, p: (k, 0, 0)),
        ],
        out_specs=pl.BlockSpec((1, BP, C), lambda k, p: (k, p, 0)),
        out_shape=jax.ShapeDtypeStruct((KV, P, C), jnp.float32),
        compiler_params=pltpu.CompilerParams(
            dimension_semantics=("parallel", "arbitrary"),
        ),
    )(gathered, w)


def _sc_compiler_params():
    cp = pltpu.CompilerParams()
    if "needs_layout_passes" in pltpu.CompilerParams.__dataclass_fields__:
        cp = dataclasses.replace(cp, needs_layout_passes=False)
    return cp


def _scatter_sc(contrib, out_idx):
    mesh = plsc.VectorSubcoreMesh(core_axis_name="c", subcore_axis_name="s")

    @functools.partial(
        pl.kernel,
        out_type=jax.ShapeDtypeStruct((N, C), jnp.float32),
        mesh=mesh,
        compiler_params=_sc_compiler_params(),
        scratch_types=[
            pltpu.VMEM((SEG,), jnp.int32),
            pltpu.VMEM((CAP_CH, CCH), jnp.int32),
            pltpu.VMEM((CAP_CH, CCH), jnp.int32),
            pltpu.VMEM((CCH, C), jnp.float32),
            pltpu.VMEM((CCH, C), jnp.float32),
            pltpu.VMEM_SHARED((ACC_ROWS, C), jnp.float32),
            pltpu.SemaphoreType.DMA,
            pltpu.SemaphoreType.DMA,
            pltpu.SemaphoreType.DMA,
            pltpu.SemaphoreType.DMA,
        ],
    )
    def k(contrib_hbm, idx_hbm, out_hbm, idxseg, loc, pos, rows_a, rows_b,
          acc, gsem_a, gsem_b, asem_a, asem_b):
        cid = lax.axis_index("c")
        sid = lax.axis_index("s")

        zero16f = jnp.zeros((16,), jnp.float32)
        zero16i = jnp.zeros((16,), jnp.int32)
        dump16 = jnp.full((16,), DUMP, jnp.int32)
        lane = lax.iota(jnp.int32, 16)

        def gather_desc(j, buf, sem):
            return pltpu.make_async_copy(contrib_hbm.at[pos.at[j]], buf, sem)

        def add_desc(j, buf, sem):
            return pltpu.make_async_copy(buf, acc.at[loc.at[j]], sem)

        def process(cntv):
            cnt = jnp.max(cntv)
            top = lax.bitwise_and(cnt + 2 * CCH - 1, -(2 * CCH))
            for gi in range(2 * CCH // 16):
                q = cnt + gi * 16 + lane
                maskp = q < top
                row_i = lax.shift_right_logical(q, 7)
                col_i = lax.bitwise_and(q, CCH - 1)
                dumpv = DUMP + lax.bitwise_and(q, 63)
                plsc.store_scatter(loc, [row_i, col_i], dumpv, mask=maskp)
                plsc.store_scatter(pos, [row_i, col_i], zero16i, mask=maskp)

            npair = lax.shift_right_logical(top, 8)

            @pl.when(npair > 0)
            def _():
                gather_desc(0, rows_a, gsem_a).start()
                gather_desc(1, rows_b, gsem_b).start()

            def chunk_pair(p, carry):
                j0 = 2 * p
                j1 = j0 + 1
                gather_desc(j0, rows_a, gsem_a).wait()
                add_desc(j0, rows_a, asem_a).start(add=True)
                gather_desc(j1, rows_b, gsem_b).wait()
                add_desc(j1, rows_b, asem_b).start(add=True)

                @pl.when(p + 1 < npair)
                def _():
                    add_desc(j0, rows_a, asem_a).wait()
                    gather_desc(j0 + 2, rows_a, gsem_a).start()
                    add_desc(j1, rows_b, asem_b).wait()
                    gather_desc(j1 + 2, rows_b, gsem_b).start()

                return carry

            lax.fori_loop(0, npair, chunk_pair, jnp.int32(0))

            @pl.when(npair > 0)
            def _():
                add_desc(0, rows_a, asem_a).wait()
                add_desc(1, rows_b, asem_b).wait()

            return jnp.zeros((16,), jnp.int32)

        for t_local in range(TPC):
            base = (TPC * cid + t_local) * TILE
            rows_t = jnp.minimum(TILE, N - base)

            @pl.loop(0, CCH)
            def _(r):
                @pl.loop(0, C, step=16)
                def _(cc):
                    rows_a[r, pl.ds(cc, 16)] = zero16f

            @pl.loop(0, ACC_ROWS // CCH)
            def _(m):
                @pl.when(lax.rem(m, NS) == sid)
                def _():
                    pltpu.sync_copy(rows_a, acc.at[pl.ds(m * CCH, CCH)])

            plsc.subcore_barrier()

            def seg_body(g, cntv):
                cntv = lax.cond(jnp.max(cntv) + SEG > CAP, process,
                                lambda c: c, cntv)
                pltpu.sync_copy(
                    idx_hbm.at[pl.ds(sid * S_ROWS + g * SEG, SEG)], idxseg)

                def scan_group(i, cntv):
                    col = i * 16
                    v = idxseg[pl.ds(col, 16)]
                    localv = v - base
                    maskv = (localv >= 0) & (localv < rows_t)
                    pc = plsc.cumsum(maskv.astype(jnp.int32))
                    q = cntv + pc - 1
                    row_i = lax.shift_right_logical(q, 7)
                    col_i = lax.bitwise_and(q, CCH - 1)
                    plsc.store_scatter(loc, [row_i, col_i], localv,
                                       mask=maskv)
                    pv = (sid * S_ROWS + g * SEG + col) + lane
                    plsc.store_scatter(pos, [row_i, col_i], pv, mask=maskv)
                    return cntv + plsc.all_reduce_population_count(maskv)

                return lax.fori_loop(0, SEG_G, scan_group, cntv)

            cntv = lax.fori_loop(0, S_ROWS // SEG, seg_body,
                                 jnp.zeros((16,), jnp.int32))
            process(cntv)

            plsc.subcore_barrier()

            mcov = rows_t - lax.rem(rows_t, WB_CH)

            @pl.loop(0, WB_N)
            def _(m):
                @pl.when((lax.rem(m, NS) == sid) & ((m + 1) * WB_CH <= rows_t))
                def _():
                    pltpu.sync_copy(acc.at[pl.ds(m * WB_CH, WB_CH)],
                                    out_hbm.at[pl.ds(base + m * WB_CH, WB_CH)])

            for mt in range(WB_CH // 16):
                @pl.when((sid == mt) & (mcov + (mt + 1) * 16 <= rows_t))
                def _():
                    pltpu.sync_copy(
                        acc.at[pl.ds(mcov + mt * 16, 16)],
                        out_hbm.at[pl.ds(base + mcov + mt * 16, 16)])

            plsc.subcore_barrier()

    return k(contrib, out_idx)


def kernel(coords, feats, maps, mappat, kernel):
    w = kernel
    in_idx = maps[:, :, 0].reshape(TOT)
    out_idx = maps[:, :, 1].reshape(TOT)
    gathered = _gather_sc(feats, in_idx)
    contrib = _gemm_tc(gathered.reshape(KV, P, C), w)
    return _scatter_sc(contrib.reshape(TOT, C), out_idx)


# speedup vs baseline: 1.5134x; 1.5134x over previous
import dataclasses
import functools

import jax
import jax.numpy as jnp
from jax import lax
from jax.experimental import pallas as pl
from jax.experimental.pallas import tpu as pltpu
from jax.experimental.pallas import tpu_sc as plsc

N = 50000
C = 128
KV = 27
P = 8192
TOT = KV * P
NC = 2
NS = 16
NW = NC * NS

G_ROWS = TOT // NW
G_CH = 128
G_NCH = G_ROWS // G_CH

TILE = 12512
TPC = 2
S_ROWS = TOT // NS
SEG = 1728
SEG_G = SEG // 16
CCH = 128
CAP = 5248
CAP_CH = CAP // CCH
DUMP = TILE
ACC_ROWS = 12544
WB_CH = 96
WB_N = TILE // WB_CH


def _gather_sc(feats, in_idx):
    mesh = plsc.VectorSubcoreMesh(core_axis_name="c", subcore_axis_name="s")

    @functools.partial(
        pl.kernel,
        out_type=jax.ShapeDtypeStruct((TOT, C), jnp.float32),
        mesh=mesh,
        scratch_types=[
            pltpu.VMEM((G_ROWS,), jnp.int32),
            pltpu.VMEM((G_CH, C), jnp.float32),
            pltpu.VMEM((G_CH, C), jnp.float32),
            pltpu.SemaphoreType.DMA,
            pltpu.SemaphoreType.DMA,
            pltpu.SemaphoreType.DMA,
            pltpu.SemaphoreType.DMA,
        ],
    )
    def k(feats_hbm, idx_hbm, out_hbm, idx_v, rows_a, rows_b,
          gsem_a, gsem_b, wsem_a, wsem_b):
        wid = lax.axis_index("s") * NC + lax.axis_index("c")
        base = wid * G_ROWS
        pltpu.sync_copy(idx_hbm.at[pl.ds(base, G_ROWS)], idx_v)

        def gather_desc(j, buf, sem):
            return pltpu.make_async_copy(
                feats_hbm.at[idx_v.at[pl.ds(j * G_CH, G_CH)]], buf, sem)

        def write_desc(j, buf, sem):
            return pltpu.make_async_copy(
                buf, out_hbm.at[pl.ds(base + j * G_CH, G_CH)], sem)

        gather_desc(0, rows_a, gsem_a).start()
        gather_desc(1, rows_b, gsem_b).start()

        def body(p, carry):
            j0 = 2 * p
            j1 = j0 + 1
            gather_desc(j0, rows_a, gsem_a).wait()
            write_desc(j0, rows_a, wsem_a).start()
            gather_desc(j1, rows_b, gsem_b).wait()
            write_desc(j1, rows_b, wsem_b).start()

            @pl.when(p + 1 < G_NCH // 2)
            def _():
                write_desc(j0, rows_a, wsem_a).wait()
                gather_desc(j0 + 2, rows_a, gsem_a).start()
                write_desc(j1, rows_b, wsem_b).wait()
                gather_desc(j1 + 2, rows_b, gsem_b).start()

            return carry

        lax.fori_loop(0, G_NCH // 2, body, jnp.int32(0))
        write_desc(G_NCH - 2, rows_a, wsem_a).wait()
        write_desc(G_NCH - 1, rows_b, wsem_b).wait()

    return k(feats, in_idx)


def _gemm_tc(gathered, w):
    BP = 4096

    def body(x_ref, w_ref, o_ref):
        x = x_ref[0].astype(jnp.bfloat16)
        wb = w_ref[0].astype(jnp.bfloat16)
        o_ref[...] = jnp.dot(x, wb, preferred_element_type=jnp.float32)[None]

    return pl.pallas_call(
        body,
        grid=(KV, P // BP),
        in_specs=[
            pl.BlockSpec((1, BP, C), lambda k, p: (k, p, 0)),
            pl.BlockSpec((1, C, C), lambda k, p: (k, 0, 0)),
        ],
        out_specs=pl.BlockSpec((1, BP, C), lambda k, p: (k, p, 0)),
        out_shape=jax.ShapeDtypeStruct((KV, P, C), jnp.float32),
        compiler_params=pltpu.CompilerParams(
            dimension_semantics=("parallel", "arbitrary"),
        ),
    )(gathered, w)


def _sc_compiler_params():
    cp = pltpu.CompilerParams()
    if "needs_layout_passes" in pltpu.CompilerParams.__dataclass_fields__:
        cp = dataclasses.replace(cp, needs_layout_passes=False)
    return cp


def _scatter_sc(contrib, out_idx):
    mesh = plsc.VectorSubcoreMesh(core_axis_name="c", subcore_axis_name="s")

    @functools.partial(
        pl.kernel,
        out_type=jax.ShapeDtypeStruct((N, C), jnp.float32),
        mesh=mesh,
        compiler_params=_sc_compiler_params(),
        scratch_types=[
            pltpu.VMEM((SEG,), jnp.int32),
            pltpu.VMEM((CAP_CH, CCH), jnp.int32),
            pltpu.VMEM((CAP_CH, CCH), jnp.int32),
            pltpu.VMEM((CCH, C), jnp.float32),
            pltpu.VMEM_SHARED((ACC_ROWS, C), jnp.float32),
            pltpu.SemaphoreType.DMA,
        ],
    )
    def k(contrib_hbm, idx_hbm, out_hbm, idxseg, loc, pos, rows_a, acc, sem):
        cid = lax.axis_index("c")
        sid = lax.axis_index("s")

        zero16f = jnp.zeros((16,), jnp.float32)
        zero16i = jnp.zeros((16,), jnp.int32)
        lane = lax.iota(jnp.int32, 16)

        def process(cntv):
            cnt = jnp.max(cntv)
            top = lax.bitwise_and(cnt + CCH - 1, -CCH)
            for gi in range(CCH // 16):
                q = cnt + gi * 16 + lane
                maskp = q < top
                row_i = lax.shift_right_logical(q, 7)
                col_i = lax.bitwise_and(q, CCH - 1)
                dumpv = DUMP + lax.bitwise_and(q, 31)
                plsc.store_scatter(loc, [row_i, col_i], dumpv, mask=maskp)
                plsc.store_scatter(pos, [row_i, col_i], zero16i, mask=maskp)

            def chunk_body(j, carry):
                pltpu.async_copy(contrib_hbm.at[pos.at[j]], rows_a,
                                 sem).wait()
                pltpu.sync_copy(rows_a, acc.at[loc.at[j]], add=True)
                return carry

            lax.fori_loop(0, lax.shift_right_logical(top, 7), chunk_body,
                          jnp.int32(0))
            return jnp.zeros((16,), jnp.int32)

        for t_local in range(TPC):
            base = (TPC * cid + t_local) * TILE
            rows_t = jnp.minimum(TILE, N - base)

            @pl.loop(0, CCH)
            def _(r):
                @pl.loop(0, C, step=16)
                def _(cc):
                    rows_a[r, pl.ds(cc, 16)] = zero16f

            @pl.loop(0, ACC_ROWS // CCH)
            def _(m):
                @pl.when(lax.rem(m, NS) == sid)
                def _():
                    pltpu.sync_copy(rows_a, acc.at[pl.ds(m * CCH, CCH)])

            plsc.subcore_barrier()

            def seg_body(g, cntv):
                cntv = lax.cond(jnp.max(cntv) + SEG > CAP, process,
                                lambda c: c, cntv)
                pltpu.sync_copy(
                    idx_hbm.at[pl.ds(sid * S_ROWS + g * SEG, SEG)], idxseg)

                def scan_group(i, cntv):
                    col = i * 16
                    v = idxseg[pl.ds(col, 16)]
                    localv = v - base
                    maskv = (localv >= 0) & (localv < rows_t)
                    pc = plsc.cumsum(maskv.astype(jnp.int32))
                    q = cntv + pc - 1
                    row_i = lax.shift_right_logical(q, 7)
                    col_i = lax.bitwise_and(q, CCH - 1)
                    plsc.store_scatter(loc, [row_i, col_i], localv,
                                       mask=maskv)
                    pv = (sid * S_ROWS + g * SEG + col) + lane
                    plsc.store_scatter(pos, [row_i, col_i], pv, mask=maskv)
                    return cntv + plsc.all_reduce_population_count(maskv)

                return lax.fori_loop(0, SEG_G, scan_group, cntv)

            cntv = lax.fori_loop(0, S_ROWS // SEG, seg_body,
                                 jnp.zeros((16,), jnp.int32))
            process(cntv)

            plsc.subcore_barrier()

            mcov = rows_t - lax.rem(rows_t, WB_CH)

            @pl.loop(0, WB_N)
            def _(m):
                @pl.when((lax.rem(m, NS) == sid) & ((m + 1) * WB_CH <= rows_t))
                def _():
                    pltpu.sync_copy(acc.at[pl.ds(m * WB_CH, WB_CH)],
                                    out_hbm.at[pl.ds(base + m * WB_CH, WB_CH)])

            for mt in range(WB_CH // 16):
                @pl.when((sid == mt) & (mcov + (mt + 1) * 16 <= rows_t))
                def _():
                    pltpu.sync_copy(
                        acc.at[pl.ds(mcov + mt * 16, 16)],
                        out_hbm.at[pl.ds(base + mcov + mt * 16, 16)])

            plsc.subcore_barrier()

    return k(contrib, out_idx)


def kernel(coords, feats, maps, mappat, kernel):
    w = kernel
    in_idx = maps[:, :, 0].reshape(TOT)
    out_idx = maps[:, :, 1].reshape(TOT)
    gathered = _gather_sc(feats, in_idx)
    contrib = _gemm_tc(gathered.reshape(KV, P, C), w)
    return _scatter_sc(contrib.reshape(TOT, C), out_idx)


# speedup vs baseline: 1.8011x; 1.1901x over previous
import dataclasses
import functools

import jax
import jax.numpy as jnp
from jax import lax
from jax.experimental import pallas as pl
from jax.experimental.pallas import tpu as pltpu
from jax.experimental.pallas import tpu_sc as plsc

N = 50000
C = 128
KV = 27
P = 8192
TOT = KV * P
NC = 2
NS = 16
NW = NC * NS

HALF = TOT // 2
G_ROWS = HALF // NW
G_CH = 128
G_NCH = G_ROWS // G_CH

TILE = 12512
TPC = 2
S_ROWS = TOT // NS
SEG = 1728
SEG_G = SEG // 16
CCH = 128
CAP = 5248
CAP_CH = CAP // CCH
DUMP = TILE
ACC_ROWS = 12544
WB_CH = 96
WB_N = TILE // WB_CH


def _gather_sc(feats, in_idx):
    mesh = plsc.VectorSubcoreMesh(core_axis_name="c", subcore_axis_name="s")

    @functools.partial(
        pl.kernel,
        out_type=jax.ShapeDtypeStruct((HALF, C), jnp.float32),
        mesh=mesh,
        scratch_types=[
            pltpu.VMEM((G_ROWS,), jnp.int32),
            pltpu.VMEM((G_CH, C), jnp.float32),
            pltpu.VMEM((G_CH, C), jnp.float32),
            pltpu.SemaphoreType.DMA,
            pltpu.SemaphoreType.DMA,
            pltpu.SemaphoreType.DMA,
            pltpu.SemaphoreType.DMA,
        ],
    )
    def k(feats_hbm, idx_hbm, out_hbm, idx_v, rows_a, rows_b,
          gsem_a, gsem_b, wsem_a, wsem_b):
        wid = lax.axis_index("s") * NC + lax.axis_index("c")
        base = wid * G_ROWS
        pltpu.sync_copy(idx_hbm.at[pl.ds(base, G_ROWS)], idx_v)

        def gather_desc(j, buf, sem):
            return pltpu.make_async_copy(
                feats_hbm.at[idx_v.at[pl.ds(j * G_CH, G_CH)]], buf, sem)

        def write_desc(j, buf, sem):
            return pltpu.make_async_copy(
                buf, out_hbm.at[pl.ds(base + j * G_CH, G_CH)], sem)

        gather_desc(0, rows_a, gsem_a).start()
        gather_desc(1, rows_b, gsem_b).start()

        def body(p, carry):
            j0 = 2 * p
            j1 = j0 + 1
            gather_desc(j0, rows_a, gsem_a).wait()
            write_desc(j0, rows_a, wsem_a).start()
            gather_desc(j1, rows_b, gsem_b).wait()
            write_desc(j1, rows_b, wsem_b).start()

            @pl.when(p + 1 < G_NCH // 2)
            def _():
                write_desc(j0, rows_a, wsem_a).wait()
                gather_desc(j0 + 2, rows_a, gsem_a).start()
                write_desc(j1, rows_b, wsem_b).wait()
                gather_desc(j1 + 2, rows_b, gsem_b).start()

            return carry

        lax.fori_loop(0, G_NCH // 2, body, jnp.int32(0))
        write_desc(G_NCH - 3, rows_a, wsem_a).wait()
        write_desc(G_NCH - 2, rows_b, wsem_b).wait()
        gather_desc(G_NCH - 1, rows_a, gsem_a).start()
        gather_desc(G_NCH - 1, rows_a, gsem_a).wait()
        write_desc(G_NCH - 1, rows_a, wsem_a).start()
        write_desc(G_NCH - 1, rows_a, wsem_a).wait()

    return k(feats, in_idx)


def _gemm_tc(gathered, w, half):
    BP = 4096
    KB = P // BP

    def body(x_ref, w_ref, o_ref):
        x = x_ref[...].astype(jnp.bfloat16)
        wb = w_ref[0].astype(jnp.bfloat16)
        o_ref[...] = jnp.dot(x, wb, preferred_element_type=jnp.float32)

    blk0 = half * (HALF // BP)
    return pl.pallas_call(
        body,
        grid=(HALF // BP,),
        in_specs=[
            pl.BlockSpec((BP, C), lambda p: (p, 0)),
            pl.BlockSpec((1, C, C), lambda p: ((blk0 + p) // KB, 0, 0)),
        ],
        out_specs=pl.BlockSpec((BP, C), lambda p: (p, 0)),
        out_shape=jax.ShapeDtypeStruct((HALF, C), jnp.float32),
        compiler_params=pltpu.CompilerParams(
            dimension_semantics=("parallel",),
        ),
    )(gathered, w)


def _sc_compiler_params():
    cp = pltpu.CompilerParams()
    if "needs_layout_passes" in pltpu.CompilerParams.__dataclass_fields__:
        cp = dataclasses.replace(cp, needs_layout_passes=False)
    return cp


def _scatter_sc(contrib_a, contrib_b, out_idx):
    mesh = plsc.VectorSubcoreMesh(core_axis_name="c", subcore_axis_name="s")

    @functools.partial(
        pl.kernel,
        out_type=jax.ShapeDtypeStruct((N, C), jnp.float32),
        mesh=mesh,
        compiler_params=_sc_compiler_params(),
        scratch_types=[
            pltpu.VMEM((SEG,), jnp.int32),
            pltpu.VMEM((CAP_CH, CCH), jnp.int32),
            pltpu.VMEM((CAP_CH, CCH), jnp.int32),
            pltpu.VMEM((CCH, C), jnp.float32),
            pltpu.VMEM_SHARED((ACC_ROWS, C), jnp.float32),
            pltpu.SemaphoreType.DMA,
        ],
    )
    def k(ca_hbm, cb_hbm, idx_hbm, out_hbm, idxseg, loc, pos, rows_a, acc,
          sem):
        cid = lax.axis_index("c")
        sid = lax.axis_index("s")
        in_b = sid >= NS // 2

        zero16f = jnp.zeros((16,), jnp.float32)
        zero16i = jnp.zeros((16,), jnp.int32)
        lane = lax.iota(jnp.int32, 16)

        def process(cntv):
            cnt = jnp.max(cntv)
            top = lax.bitwise_and(cnt + CCH - 1, -CCH)
            for gi in range(CCH // 16):
                q = cnt + gi * 16 + lane
                maskp = q < top
                row_i = lax.shift_right_logical(q, 7)
                col_i = lax.bitwise_and(q, CCH - 1)
                dumpv = DUMP + lax.bitwise_and(q, 31)
                plsc.store_scatter(loc, [row_i, col_i], dumpv, mask=maskp)
                plsc.store_scatter(pos, [row_i, col_i], zero16i, mask=maskp)

            def chunk_body(j, carry):
                @pl.when(in_b)
                def _():
                    pltpu.async_copy(cb_hbm.at[pos.at[j]], rows_a, sem).wait()

                @pl.when(jnp.logical_not(in_b))
                def _():
                    pltpu.async_copy(ca_hbm.at[pos.at[j]], rows_a, sem).wait()

                pltpu.sync_copy(rows_a, acc.at[loc.at[j]], add=True)
                return carry

            lax.fori_loop(0, lax.shift_right_logical(top, 7), chunk_body,
                          jnp.int32(0))
            return jnp.zeros((16,), jnp.int32)

        for t_local in range(TPC):
            base = (TPC * cid + t_local) * TILE
            rows_t = jnp.minimum(TILE, N - base)

            @pl.loop(0, CCH)
            def _(r):
                @pl.loop(0, C, step=16)
                def _(cc):
                    rows_a[r, pl.ds(cc, 16)] = zero16f

            @pl.loop(0, ACC_ROWS // CCH)
            def _(m):
                @pl.when(lax.rem(m, NS) == sid)
                def _():
                    pltpu.sync_copy(rows_a, acc.at[pl.ds(m * CCH, CCH)])

            plsc.subcore_barrier()

            def seg_body(g, cntv):
                cntv = lax.cond(jnp.max(cntv) + SEG > CAP, process,
                                lambda c: c, cntv)
                pltpu.sync_copy(
                    idx_hbm.at[pl.ds(sid * S_ROWS + g * SEG, SEG)], idxseg)

                def scan_group(i, cntv):
                    col = i * 16
                    v = idxseg[pl.ds(col, 16)]
                    localv = v - base
                    maskv = (localv >= 0) & (localv < rows_t)
                    pc = plsc.cumsum(maskv.astype(jnp.int32))
                    q = cntv + pc - 1
                    row_i = lax.shift_right_logical(q, 7)
                    col_i = lax.bitwise_and(q, CCH - 1)
                    plsc.store_scatter(loc, [row_i, col_i], localv,
                                       mask=maskv)
                    pv = (lax.rem(sid, NS // 2) * S_ROWS + g * SEG + col) + lane
                    plsc.store_scatter(pos, [row_i, col_i], pv, mask=maskv)
                    return cntv + plsc.all_reduce_population_count(maskv)

                return lax.fori_loop(0, SEG_G, scan_group, cntv)

            cntv = lax.fori_loop(0, S_ROWS // SEG, seg_body,
                                 jnp.zeros((16,), jnp.int32))
            process(cntv)

            plsc.subcore_barrier()

            mcov = rows_t - lax.rem(rows_t, WB_CH)

            @pl.loop(0, WB_N)
            def _(m):
                @pl.when((lax.rem(m, NS) == sid) & ((m + 1) * WB_CH <= rows_t))
                def _():
                    pltpu.sync_copy(acc.at[pl.ds(m * WB_CH, WB_CH)],
                                    out_hbm.at[pl.ds(base + m * WB_CH, WB_CH)])

            for mt in range(WB_CH // 16):
                @pl.when((sid == mt) & (mcov + (mt + 1) * 16 <= rows_t))
                def _():
                    pltpu.sync_copy(
                        acc.at[pl.ds(mcov + mt * 16, 16)],
                        out_hbm.at[pl.ds(base + mcov + mt * 16, 16)])

            plsc.subcore_barrier()

    return k(contrib_a, contrib_b, out_idx)


def kernel(coords, feats, maps, mappat, kernel):
    w = kernel
    in_idx = maps[:, :, 0].reshape(TOT)
    out_idx = maps[:, :, 1].reshape(TOT)
    gathered_a = _gather_sc(feats, in_idx[:HALF])
    contrib_a = _gemm_tc(gathered_a, w, 0)
    gathered_b = _gather_sc(feats, in_idx[HALF:])
    contrib_b = _gemm_tc(gathered_b, w, 1)
    return _scatter_sc(contrib_a, contrib_b, out_idx)


# speedup vs baseline: 1.8888x; 1.0487x over previous
import dataclasses
import functools

import jax
import jax.numpy as jnp
from jax import lax
from jax.experimental import pallas as pl
from jax.experimental.pallas import tpu as pltpu
from jax.experimental.pallas import tpu_sc as plsc

N = 50000
C = 128
KV = 27
P = 8192
TOT = KV * P
NC = 2
NS = 16
NW = NC * NS

NQ = 4
QTR = TOT // NQ
SPQ = NS // NQ
G_ROWS = QTR // NW
G_CH = 64
G_NCH = G_ROWS // G_CH

TILE = 12512
TPC = 2
S_ROWS = TOT // NS
SEG = 1728
SEG_G = SEG // 16
CCH = 128
CAP = 5248
CAP_CH = CAP // CCH
DUMP = TILE
ACC_ROWS = 12544
WB_CH = 96
WB_N = TILE // WB_CH


def _gather_sc(feats, in_idx):
    mesh = plsc.VectorSubcoreMesh(core_axis_name="c", subcore_axis_name="s")

    @functools.partial(
        pl.kernel,
        out_type=jax.ShapeDtypeStruct((QTR, C), jnp.float32),
        mesh=mesh,
        scratch_types=[
            pltpu.VMEM((G_ROWS,), jnp.int32),
            pltpu.VMEM((G_CH, C), jnp.float32),
            pltpu.VMEM((G_CH, C), jnp.float32),
            pltpu.SemaphoreType.DMA,
            pltpu.SemaphoreType.DMA,
            pltpu.SemaphoreType.DMA,
            pltpu.SemaphoreType.DMA,
        ],
    )
    def k(feats_hbm, idx_hbm, out_hbm, idx_v, rows_a, rows_b,
          gsem_a, gsem_b, wsem_a, wsem_b):
        wid = lax.axis_index("s") * NC + lax.axis_index("c")
        base = wid * G_ROWS
        pltpu.sync_copy(idx_hbm.at[pl.ds(base, G_ROWS)], idx_v)

        def gather_desc(j, buf, sem):
            return pltpu.make_async_copy(
                feats_hbm.at[idx_v.at[pl.ds(j * G_CH, G_CH)]], buf, sem)

        def write_desc(j, buf, sem):
            return pltpu.make_async_copy(
                buf, out_hbm.at[pl.ds(base + j * G_CH, G_CH)], sem)

        gather_desc(0, rows_a, gsem_a).start()
        gather_desc(1, rows_b, gsem_b).start()

        def body(p, carry):
            j0 = 2 * p
            j1 = j0 + 1
            gather_desc(j0, rows_a, gsem_a).wait()
            write_desc(j0, rows_a, wsem_a).start()
            gather_desc(j1, rows_b, gsem_b).wait()
            write_desc(j1, rows_b, wsem_b).start()

            @pl.when(p + 1 < G_NCH // 2)
            def _():
                write_desc(j0, rows_a, wsem_a).wait()
                gather_desc(j0 + 2, rows_a, gsem_a).start()
                write_desc(j1, rows_b, wsem_b).wait()
                gather_desc(j1 + 2, rows_b, gsem_b).start()

            return carry

        lax.fori_loop(0, G_NCH // 2, body, jnp.int32(0))
        write_desc(G_NCH - 3, rows_a, wsem_a).wait()
        write_desc(G_NCH - 2, rows_b, wsem_b).wait()
        gather_desc(G_NCH - 1, rows_a, gsem_a).start()
        gather_desc(G_NCH - 1, rows_a, gsem_a).wait()
        write_desc(G_NCH - 1, rows_a, wsem_a).start()
        write_desc(G_NCH - 1, rows_a, wsem_a).wait()

    return k(feats, in_idx)


def _gemm_tc(gathered, w, quarter):
    BP = 2048
    KB = P // BP

    def body(x_ref, w_ref, o_ref):
        x = x_ref[...].astype(jnp.bfloat16)
        wb = w_ref[0].astype(jnp.bfloat16)
        o_ref[...] = jnp.dot(x, wb, preferred_element_type=jnp.float32)

    blk0 = quarter * (QTR // BP)
    return pl.pallas_call(
        body,
        grid=(QTR // BP,),
        in_specs=[
            pl.BlockSpec((BP, C), lambda p: (p, 0)),
            pl.BlockSpec((1, C, C), lambda p: ((blk0 + p) // KB, 0, 0)),
        ],
        out_specs=pl.BlockSpec((BP, C), lambda p: (p, 0)),
        out_shape=jax.ShapeDtypeStruct((QTR, C), jnp.float32),
        compiler_params=pltpu.CompilerParams(
            dimension_semantics=("parallel",),
        ),
    )(gathered, w)


def _sc_compiler_params():
    cp = pltpu.CompilerParams()
    if "needs_layout_passes" in pltpu.CompilerParams.__dataclass_fields__:
        cp = dataclasses.replace(cp, needs_layout_passes=False)
    return cp


def _scatter_sc(contribs, out_idx):
    mesh = plsc.VectorSubcoreMesh(core_axis_name="c", subcore_axis_name="s")

    @functools.partial(
        pl.kernel,
        out_type=jax.ShapeDtypeStruct((N, C), jnp.float32),
        mesh=mesh,
        compiler_params=_sc_compiler_params(),
        scratch_types=[
            pltpu.VMEM((SEG,), jnp.int32),
            pltpu.VMEM((CAP_CH, CCH), jnp.int32),
            pltpu.VMEM((CAP_CH, CCH), jnp.int32),
            pltpu.VMEM((CCH, C), jnp.float32),
            pltpu.VMEM_SHARED((ACC_ROWS, C), jnp.float32),
            pltpu.SemaphoreType.DMA,
        ],
    )
    def k(c0_hbm, c1_hbm, c2_hbm, c3_hbm, idx_hbm, out_hbm, idxseg, loc, pos,
          rows_a, acc, sem):
        cid = lax.axis_index("c")
        sid = lax.axis_index("s")
        qid = lax.div(sid, SPQ)
        c_hbms = [c0_hbm, c1_hbm, c2_hbm, c3_hbm]

        zero16f = jnp.zeros((16,), jnp.float32)
        zero16i = jnp.zeros((16,), jnp.int32)
        lane = lax.iota(jnp.int32, 16)

        def process(cntv):
            cnt = jnp.max(cntv)
            top = lax.bitwise_and(cnt + CCH - 1, -CCH)
            for gi in range(CCH // 16):
                q = cnt + gi * 16 + lane
                maskp = q < top
                row_i = lax.shift_right_logical(q, 7)
                col_i = lax.bitwise_and(q, CCH - 1)
                dumpv = DUMP + lax.bitwise_and(q, 31)
                plsc.store_scatter(loc, [row_i, col_i], dumpv, mask=maskp)
                plsc.store_scatter(pos, [row_i, col_i], zero16i, mask=maskp)

            def chunk_body(j, carry):
                for qq in range(NQ):
                    @pl.when(qid == qq)
                    def _():
                        pltpu.async_copy(c_hbms[qq].at[pos.at[j]], rows_a,
                                         sem).wait()

                pltpu.sync_copy(rows_a, acc.at[loc.at[j]], add=True)
                return carry

            lax.fori_loop(0, lax.shift_right_logical(top, 7), chunk_body,
                          jnp.int32(0))
            return jnp.zeros((16,), jnp.int32)

        for t_local in range(TPC):
            base = (TPC * cid + t_local) * TILE
            rows_t = jnp.minimum(TILE, N - base)

            @pl.loop(0, CCH)
            def _(r):
                @pl.loop(0, C, step=16)
                def _(cc):
                    rows_a[r, pl.ds(cc, 16)] = zero16f

            @pl.loop(0, ACC_ROWS // CCH)
            def _(m):
                @pl.when(lax.rem(m, NS) == sid)
                def _():
                    pltpu.sync_copy(rows_a, acc.at[pl.ds(m * CCH, CCH)])

            plsc.subcore_barrier()

            def seg_body(g, cntv):
                cntv = lax.cond(jnp.max(cntv) + SEG > CAP, process,
                                lambda c: c, cntv)
                pltpu.sync_copy(
                    idx_hbm.at[pl.ds(sid * S_ROWS + g * SEG, SEG)], idxseg)

                def scan_group(i, cntv):
                    col = i * 16
                    v = idxseg[pl.ds(col, 16)]
                    localv = v - base
                    maskv = (localv >= 0) & (localv < rows_t)
                    pc = plsc.cumsum(maskv.astype(jnp.int32))
                    q = cntv + pc - 1
                    row_i = lax.shift_right_logical(q, 7)
                    col_i = lax.bitwise_and(q, CCH - 1)
                    plsc.store_scatter(loc, [row_i, col_i], localv,
                                       mask=maskv)
                    pv = (lax.rem(sid, SPQ) * S_ROWS + g * SEG + col) + lane
                    plsc.store_scatter(pos, [row_i, col_i], pv, mask=maskv)
                    return cntv + plsc.all_reduce_population_count(maskv)

                return lax.fori_loop(0, SEG_G, scan_group, cntv)

            cntv = lax.fori_loop(0, S_ROWS // SEG, seg_body,
                                 jnp.zeros((16,), jnp.int32))
            process(cntv)

            plsc.subcore_barrier()

            mcov = rows_t - lax.rem(rows_t, WB_CH)

            @pl.loop(0, WB_N)
            def _(m):
                @pl.when((lax.rem(m, NS) == sid) & ((m + 1) * WB_CH <= rows_t))
                def _():
                    pltpu.sync_copy(acc.at[pl.ds(m * WB_CH, WB_CH)],
                                    out_hbm.at[pl.ds(base + m * WB_CH, WB_CH)])

            for mt in range(WB_CH // 16):
                @pl.when((sid == mt) & (mcov + (mt + 1) * 16 <= rows_t))
                def _():
                    pltpu.sync_copy(
                        acc.at[pl.ds(mcov + mt * 16, 16)],
                        out_hbm.at[pl.ds(base + mcov + mt * 16, 16)])

            plsc.subcore_barrier()

    return k(*contribs, out_idx)


def kernel(coords, feats, maps, mappat, kernel):
    w = kernel
    in_idx = maps[:, :, 0].reshape(TOT)
    out_idx = maps[:, :, 1].reshape(TOT)
    contribs = []
    for q in range(NQ):
        gathered = _gather_sc(feats, in_idx[q * QTR:(q + 1) * QTR])
        contribs.append(_gemm_tc(gathered, w, q))
    return _scatter_sc(contribs, out_idx)
